# baseline (speedup 1.0000x reference)
import functools

import jax
import jax.numpy as jnp
from jax import lax
from jax.experimental import pallas as pl
from jax.experimental.pallas import tpu as pltpu
from jax.experimental.pallas import tpu_sc as plsc

D = 1024
HALF = 512
MAX_GEN = 20
N_TOKENS = 8192
NW = 32
TPW = N_TOKENS // NW
CHUNK = 16
N_CHUNKS = TPW // CHUNK
NBUF = 4
N_GROUPS = N_CHUNKS // NBUF
GRP = 16
LANES = 16
SLICES = D // LANES
HSLICES = HALF // LANES


def _sc_encode(xf, gi, by, tab, wp):
    mesh = plsc.VectorSubcoreMesh(core_axis_name="c", subcore_axis_name="s")

    @functools.partial(
        pl.kernel,
        mesh=mesh,
        out_type=jax.ShapeDtypeStruct((N_TOKENS, D), jnp.float32),
        scratch_types=[
            pltpu.VMEM((MAX_GEN, D), jnp.float32),
            pltpu.VMEM((HALF,), jnp.float32),
            pltpu.VMEM((TPW + LANES,), jnp.int32),
            pltpu.VMEM((TPW + LANES,), jnp.float32),
            pltpu.VMEM((NBUF, CHUNK, D), jnp.float32),
            pltpu.SemaphoreType.DMA((NBUF,)),
            pltpu.SemaphoreType.DMA((NBUF,)),
        ],
    )
    def k(x_hbm, gi_hbm, by_hbm, tab_hbm, wp_hbm, out_hbm,
          tab_v, wp_v, gic, nyc, xr, sx, so):
        wid = lax.axis_index("s") * 2 + lax.axis_index("c")
        base = wid * TPW

        HC = CHUNK // 2

        def issue_in(c, b):
            t = base + c * CHUNK
            pltpu.async_copy(x_hbm.at[pl.ds(t, HC)],
                             xr.at[b, pl.ds(0, HC)], sx.at[b])
            pltpu.async_copy(x_hbm.at[pl.ds(t + HC, HC)],
                             xr.at[b, pl.ds(HC, HC)], sx.at[b])

        issue_in(0, 0)
        issue_in(1, 1)
        issue_in(2, 2)
        stg = [
            pltpu.async_copy(tab_hbm, tab_v, so.at[0]),
            pltpu.async_copy(wp_hbm, wp_v, so.at[1]),
            pltpu.async_copy(gi_hbm.at[pl.ds(base, TPW)],
                             gic.at[pl.ds(0, TPW)], so.at[2]),
            pltpu.async_copy(by_hbm.at[pl.ds(base, TPW)],
                             nyc.at[pl.ds(0, TPW)], so.at[3]),
        ]
        for cp in stg:
            cp.wait()
        for s in range(TPW // LANES):
            sl = pl.ds(s * LANES, LANES)
            nyc[sl] = (nyc[sl] - 1900.0) * 0.01

        def group(g, _):
            for b in range(NBUF):
                c = g * NBUF + b
                pltpu.make_async_copy(x_hbm.at[pl.ds(base, CHUNK)],
                                      xr.at[b], sx.at[b]).wait()

                for t0 in range(0, CHUNK, GRP):
                    gids = []
                    ny16s = []
                    for i in range(t0, t0 + GRP):
                        gw = gic[pl.ds(c * CHUNK + i, LANES)]
                        gids.append(gw[0])
                        nyw = nyc[pl.ds(c * CHUNK + i, LANES)]
                        ny16s.append(jnp.full((LANES,), nyw[0], jnp.float32))

                    @plsc.parallel_loop(0, HSLICES, unroll=2)
                    def _lo(j):
                        sl = pl.ds(j * LANES, LANES)
                        w = wp_v[sl]
                        for i in range(GRP):
                            plsc.addupdate(
                                xr.at[b, t0 + i, sl],
                                tab_v[gids[i], sl] + ny16s[i] * w)

                    @plsc.parallel_loop(HSLICES, SLICES, unroll=2)
                    def _hi(j):
                        sl = pl.ds(j * LANES, LANES)
                        for i in range(GRP):
                            plsc.addupdate(xr.at[b, t0 + i, sl],
                                           tab_v[gids[i], sl])

                tout = base + c * CHUNK
                pltpu.async_copy(xr.at[b, pl.ds(0, HC)],
                                 out_hbm.at[pl.ds(tout, HC)], so.at[b])
                pltpu.async_copy(xr.at[b, pl.ds(HC, HC)],
                                 out_hbm.at[pl.ds(tout + HC, HC)], so.at[b])
                pb = (b + 3) % NBUF
                cp = c + 3

                @pl.when(cp < N_CHUNKS)
                def _():
                    @pl.when(c >= 1)
                    def _():
                        pltpu.make_async_copy(
                            xr.at[pb], out_hbm.at[pl.ds(base, CHUNK)],
                            so.at[pb]).wait()
                    issue_in(cp, pb)

            return 0

        lax.fori_loop(0, N_GROUPS, group, 0)
        for b in range(NBUF):
            pltpu.make_async_copy(xr.at[b], out_hbm.at[pl.ds(base, CHUNK)],
                                  so.at[b]).wait()

    return k(xf, gi, by, tab, wp)


def kernel(x, generation_info, birth_years, gen_table, temporal_W, temporal_b):
    B, L, d = x.shape
    xf = x.reshape(B * L, d)
    gi = generation_info.reshape(-1).astype(jnp.int32)
    by = birth_years.reshape(-1)
    bp = jnp.pad(temporal_b, (0, d - temporal_b.shape[0]))
    tab = gen_table + bp[None, :]
    wp = temporal_W[:, 0]
    out = _sc_encode(xf, gi, by, tab, wp)
    return out.reshape(B, L, d)


# speedup vs baseline: 1.0158x; 1.0158x over previous
import functools

import jax
import jax.numpy as jnp
from jax import lax
from jax.experimental import pallas as pl
from jax.experimental.pallas import tpu as pltpu
from jax.experimental.pallas import tpu_sc as plsc

D = 1024
HALF = 512
MAX_GEN = 20
N_TOKENS = 8192
NW = 32
TPW = N_TOKENS // NW
CHUNK = 16
N_CHUNKS = TPW // CHUNK
NBUF = 4
N_GROUPS = N_CHUNKS // NBUF
GRP = 8
LANES = 16
SLICES = D // LANES
HSLICES = HALF // LANES


def _sc_encode(xf, gi, by, tab, wp):
    mesh = plsc.VectorSubcoreMesh(core_axis_name="c", subcore_axis_name="s")

    @functools.partial(
        pl.kernel,
        mesh=mesh,
        out_type=jax.ShapeDtypeStruct((N_TOKENS, D), jnp.float32),
        scratch_types=[
            pltpu.VMEM((MAX_GEN, D), jnp.float32),
            pltpu.VMEM((HALF,), jnp.float32),
            pltpu.VMEM((TPW + LANES,), jnp.int32),
            pltpu.VMEM((TPW + LANES,), jnp.float32),
            pltpu.VMEM((NBUF, CHUNK, D), jnp.float32),
            pltpu.SemaphoreType.DMA((NBUF,)),
            pltpu.SemaphoreType.DMA((NBUF,)),
        ],
    )
    def k(x_hbm, gi_hbm, by_hbm, tab_hbm, wp_hbm, out_hbm,
          tab_v, wp_v, gic, nyc, xr, sx, so):
        wid = lax.axis_index("s") * 2 + lax.axis_index("c")
        base = wid * TPW

        HC = CHUNK // 2

        def issue_in(c, b):
            t = base + c * CHUNK
            pltpu.async_copy(x_hbm.at[pl.ds(t, HC)],
                             xr.at[b, pl.ds(0, HC)], sx.at[b])
            pltpu.async_copy(x_hbm.at[pl.ds(t + HC, HC)],
                             xr.at[b, pl.ds(HC, HC)], sx.at[b])

        issue_in(0, 0)
        issue_in(1, 1)
        issue_in(2, 2)
        stg = [
            pltpu.async_copy(tab_hbm, tab_v, so.at[0]),
            pltpu.async_copy(wp_hbm, wp_v, so.at[1]),
            pltpu.async_copy(gi_hbm.at[pl.ds(base, TPW)],
                             gic.at[pl.ds(0, TPW)], so.at[2]),
            pltpu.async_copy(by_hbm.at[pl.ds(base, TPW)],
                             nyc.at[pl.ds(0, TPW)], so.at[3]),
        ]
        for cp in stg:
            cp.wait()
        for s in range(TPW // LANES):
            sl = pl.ds(s * LANES, LANES)
            nyc[sl] = (nyc[sl] - 1900.0) * 0.01

        def group(g, _):
            for b in range(NBUF):
                c = g * NBUF + b
                pltpu.make_async_copy(x_hbm.at[pl.ds(base, CHUNK)],
                                      xr.at[b], sx.at[b]).wait()

                for t0 in range(0, CHUNK, GRP):
                    gids = []
                    ny16s = []
                    for i in range(t0, t0 + GRP):
                        gw = gic[pl.ds(c * CHUNK + i, LANES)]
                        gids.append(gw[0])
                        nyw = nyc[pl.ds(c * CHUNK + i, LANES)]
                        ny16s.append(jnp.full((LANES,), nyw[0], jnp.float32))

                    @plsc.parallel_loop(0, HSLICES, unroll=2)
                    def _lo(j):
                        sl = pl.ds(j * LANES, LANES)
                        w = wp_v[sl]
                        for i in range(GRP):
                            plsc.addupdate(
                                xr.at[b, t0 + i, sl],
                                tab_v[gids[i], sl] + ny16s[i] * w)

                    @plsc.parallel_loop(HSLICES, SLICES, unroll=2)
                    def _hi(j):
                        sl = pl.ds(j * LANES, LANES)
                        for i in range(GRP):
                            plsc.addupdate(xr.at[b, t0 + i, sl],
                                           tab_v[gids[i], sl])

                tout = base + c * CHUNK
                pltpu.async_copy(xr.at[b, pl.ds(0, HC)],
                                 out_hbm.at[pl.ds(tout, HC)], so.at[b])
                pltpu.async_copy(xr.at[b, pl.ds(HC, HC)],
                                 out_hbm.at[pl.ds(tout + HC, HC)], so.at[b])
                pb = (b + 3) % NBUF
                cp = c + 3

                @pl.when(cp < N_CHUNKS)
                def _():
                    @pl.when(c >= 1)
                    def _():
                        pltpu.make_async_copy(
                            xr.at[pb], out_hbm.at[pl.ds(base, CHUNK)],
                            so.at[pb]).wait()
                    issue_in(cp, pb)

            return 0

        lax.fori_loop(0, N_GROUPS, group, 0)
        for b in range(NBUF):
            pltpu.make_async_copy(xr.at[b], out_hbm.at[pl.ds(base, CHUNK)],
                                  so.at[b]).wait()

    return k(xf, gi, by, tab, wp)


def kernel(x, generation_info, birth_years, gen_table, temporal_W, temporal_b):
    B, L, d = x.shape
    xf = x.reshape(B * L, d)
    gi = generation_info.reshape(-1).astype(jnp.int32)
    by = birth_years.reshape(-1)
    bp = jnp.pad(temporal_b, (0, d - temporal_b.shape[0]))
    tab = gen_table + bp[None, :]
    wp = temporal_W[:, 0]
    out = _sc_encode(xf, gi, by, tab, wp)
    return out.reshape(B, L, d)
